# baseline (speedup 1.0000x reference)
import functools

import jax
import jax.numpy as jnp
from jax import lax
from jax.experimental import pallas as pl
from jax.experimental.pallas import tpu as pltpu
from jax.experimental.pallas import tpu_sc as plsc

N_TOTAL = 16777216
NW = 32
L = 16
PER_W = N_TOTAL // NW
CHUNK = 16384
NCHUNK = PER_W // CHUNK
UNROLL = 8
TBL = 32


def _sc_body(x_hbm, knots_hbm, a_hbm, b_hbm, c_hbm, d_hbm, out_hbm,
             a_v, b_v, c_v, d_v, kn_v, pab_v, pcd_v,
             x0, x1, o0, o1, si0, si1, so0, so1):
    wid = lax.axis_index("s") * 2 + lax.axis_index("c")
    base = wid * PER_W
    xbufs, obufs = (x0, x1), (o0, o1)
    sins, souts = (si0, si1), (so0, so1)

    pltpu.sync_copy(a_hbm, a_v)
    pltpu.sync_copy(b_hbm, b_v)
    pltpu.sync_copy(c_hbm, c_v)
    pltpu.sync_copy(d_hbm, d_v)
    pltpu.sync_copy(knots_hbm.at[pl.ds(0, L)], kn_v)

    for half in (0, 1):
        ah = a_v[pl.ds(half * L, L)]
        bh = b_v[pl.ds(half * L, L)]
        pk = plsc.pack(ah, bh, format=plsc.PackFormat.INTERLEAVED)
        pab_v[pl.ds(half * L, L)] = plsc.bitcast(pk, jnp.int32)
        ch = c_v[pl.ds(half * L, L)]
        dh = d_v[pl.ds(half * L, L)]
        pk = plsc.pack(ch, dh, format=plsc.PackFormat.INTERLEAVED)
        pcd_v[pl.ds(half * L, L)] = plsc.bitcast(pk, jnp.int32)

    kv = kn_v[pl.ds(0, L)]
    k0_v = jnp.full((L,), kv[0], dtype=jnp.float32)
    k1_v = jnp.full((L,), kv[1], dtype=jnp.float32)
    dx_v = k1_v - k0_v
    inv_v = jnp.full((L,), 1.0, dtype=jnp.float32) / dx_v
    offs_v = jnp.zeros((L,), dtype=jnp.float32) - k0_v * inv_v
    big_v = jnp.full((L,), 12582912.0, dtype=jnp.float32)
    shift_v = jnp.zeros((L,), dtype=jnp.float32) - k0_v - 0.5 * dx_v
    topf_v = big_v + float(TBL - 1)
    mask_v = jnp.full((L,), TBL - 1, dtype=jnp.int32)

    def hslice(c):
        return pl.ds(pl.multiple_of(base + c * CHUNK, 8), CHUNK)

    def in_start(c, b):
        pltpu.make_async_copy(x_hbm.at[hslice(c)], xbufs[b], sins[b]).start()

    def in_wait(b):
        pltpu.make_async_copy(x_hbm.at[pl.ds(0, CHUNK)], xbufs[b],
                              sins[b]).wait()

    def out_start(c, b):
        pltpu.make_async_copy(obufs[b], out_hbm.at[hslice(c)],
                              souts[b]).start()

    def out_wait(b):
        pltpu.make_async_copy(obufs[b], out_hbm.at[pl.ds(0, CHUNK)],
                              souts[b]).wait()

    def compute(b):
        xv, ov = xbufs[b], obufs[b]

        @plsc.parallel_loop(0, CHUNK // L, unroll=UNROLL)
        def _(i):
            sl = pl.ds(i * L, L)
            xx = xv[sl]
            t = jnp.minimum(
                jnp.maximum((xx + shift_v) * inv_v + big_v, big_v), topf_v)
            idx = plsc.bitcast(t, jnp.int32) & mask_v
            pw1 = plsc.load_gather(pab_v, [idx])
            aa, bb = plsc.unpack(plsc.bitcast(pw1, jnp.bfloat16),
                                 format=plsc.PackFormat.INTERLEAVED)
            pw2 = plsc.load_gather(pcd_v, [idx])
            cc, dd = plsc.unpack(plsc.bitcast(pw2, jnp.bfloat16),
                                 format=plsc.PackFormat.INTERLEAVED)
            ov[sl] = aa + xx * (bb + xx * (cc + xx * dd))

    in_start(0, 0)
    in_start(1, 1)
    in_wait(0); compute(0); out_start(0, 0); in_start(2, 0)
    in_wait(1); compute(1); out_start(1, 1); in_start(3, 1)

    def pair(g, _):
        c0 = g * 2
        for b in (0, 1):
            in_wait(b)
            out_wait(b)
            compute(b)
            out_start(c0 + b, b)
            in_start(c0 + b + 2, b)
        return 0

    lax.fori_loop(1, NCHUNK // 2 - 1, pair, 0)

    for b in (0, 1):
        in_wait(b)
        out_wait(b)
        compute(b)
        out_start(NCHUNK - 2 + b, b)
    out_wait(0)
    out_wait(1)


_sc_kernel = functools.partial(
    pl.kernel,
    mesh=plsc.VectorSubcoreMesh(core_axis_name="c", subcore_axis_name="s"),
    out_type=jax.ShapeDtypeStruct((N_TOTAL,), jnp.float32),
    compiler_params=pltpu.CompilerParams(needs_layout_passes=False),
    scratch_types=[
        pltpu.VMEM((TBL,), jnp.float32),
        pltpu.VMEM((TBL,), jnp.float32),
        pltpu.VMEM((TBL,), jnp.float32),
        pltpu.VMEM((TBL,), jnp.float32),
        pltpu.VMEM((L,), jnp.float32),
        pltpu.VMEM((TBL,), jnp.int32),
        pltpu.VMEM((TBL,), jnp.int32),
        pltpu.VMEM((CHUNK,), jnp.float32),
        pltpu.VMEM((CHUNK,), jnp.float32),
        pltpu.VMEM((CHUNK,), jnp.float32),
        pltpu.VMEM((CHUNK,), jnp.float32),
        pltpu.SemaphoreType.DMA,
        pltpu.SemaphoreType.DMA,
        pltpu.SemaphoreType.DMA,
        pltpu.SemaphoreType.DMA,
    ],
)(_sc_body)


def kernel(x, knots, a, b, c, d):
    return _sc_kernel(x, knots, a, b, c, d)


# speedup vs baseline: 1.1020x; 1.1020x over previous
import functools

import jax
import jax.numpy as jnp
from jax import lax
from jax.experimental import pallas as pl
from jax.experimental.pallas import tpu as pltpu
from jax.experimental.pallas import tpu_sc as plsc

N_TOTAL = 16777216
NW = 32
L = 16
PER_W = N_TOTAL // NW
CHUNK = 16384
NCHUNK = PER_W // CHUNK
UNROLL = 8
TBL = 32


def _sc_body(x_hbm, knots_hbm, a_hbm, b_hbm, c_hbm, d_hbm, out_hbm,
             a_v, b_v, c_v, d_v, kn_v, pcd_v,
             x0, x1, o0, o1, si0, si1, so0, so1):
    wid = lax.axis_index("s") * 2 + lax.axis_index("c")
    base = wid * PER_W
    xbufs, obufs = (x0, x1), (o0, o1)
    sins, souts = (si0, si1), (so0, so1)

    pltpu.sync_copy(a_hbm, a_v)
    pltpu.sync_copy(b_hbm, b_v)
    pltpu.sync_copy(c_hbm, c_v)
    pltpu.sync_copy(d_hbm, d_v)
    pltpu.sync_copy(knots_hbm.at[pl.ds(0, L)], kn_v)

    for half in (0, 1):
        ch = c_v[pl.ds(half * L, L)]
        dh = d_v[pl.ds(half * L, L)]
        pk = plsc.pack(ch, dh, format=plsc.PackFormat.INTERLEAVED)
        pcd_v[pl.ds(half * L, L)] = plsc.bitcast(pk, jnp.int32)

    kv = kn_v[pl.ds(0, L)]
    k0_v = jnp.full((L,), kv[0], dtype=jnp.float32)
    k1_v = jnp.full((L,), kv[1], dtype=jnp.float32)
    dx_v = k1_v - k0_v
    inv_v = jnp.full((L,), 1.0, dtype=jnp.float32) / dx_v
    offs_v = jnp.zeros((L,), dtype=jnp.float32) - k0_v * inv_v
    big_v = jnp.full((L,), 12582912.0, dtype=jnp.float32)
    shift_v = jnp.zeros((L,), dtype=jnp.float32) - k0_v - 0.5 * dx_v
    topf_v = big_v + float(TBL - 1)
    mask_v = jnp.full((L,), TBL - 1, dtype=jnp.int32)

    def hslice(c):
        return pl.ds(pl.multiple_of(base + c * CHUNK, 8), CHUNK)

    def in_start(c, b):
        pltpu.make_async_copy(x_hbm.at[hslice(c)], xbufs[b], sins[b]).start()

    def in_wait(b):
        pltpu.make_async_copy(x_hbm.at[pl.ds(0, CHUNK)], xbufs[b],
                              sins[b]).wait()

    def out_start(c, b):
        pltpu.make_async_copy(obufs[b], out_hbm.at[hslice(c)],
                              souts[b]).start()

    def out_wait(b):
        pltpu.make_async_copy(obufs[b], out_hbm.at[pl.ds(0, CHUNK)],
                              souts[b]).wait()

    def compute(b):
        xv, ov = xbufs[b], obufs[b]

        @plsc.parallel_loop(0, CHUNK // L, unroll=UNROLL)
        def _(i):
            sl = pl.ds(i * L, L)
            xx = xv[sl]
            t = jnp.minimum(
                jnp.maximum((xx + shift_v) * inv_v + big_v, big_v), topf_v)
            idx = plsc.bitcast(t, jnp.int32) & mask_v
            aa = plsc.load_gather(a_v, [idx])
            bb = plsc.load_gather(b_v, [idx])
            pw2 = plsc.load_gather(pcd_v, [idx])
            cc, dd = plsc.unpack(plsc.bitcast(pw2, jnp.bfloat16),
                                 format=plsc.PackFormat.INTERLEAVED)
            ov[sl] = aa + xx * (bb + xx * (cc + xx * dd))

    in_start(0, 0)
    in_start(1, 1)
    in_wait(0); compute(0); out_start(0, 0); in_start(2, 0)
    in_wait(1); compute(1); out_start(1, 1); in_start(3, 1)

    def pair(g, _):
        c0 = g * 2
        for b in (0, 1):
            in_wait(b)
            out_wait(b)
            compute(b)
            out_start(c0 + b, b)
            in_start(c0 + b + 2, b)
        return 0

    lax.fori_loop(1, NCHUNK // 2 - 1, pair, 0)

    for b in (0, 1):
        in_wait(b)
        out_wait(b)
        compute(b)
        out_start(NCHUNK - 2 + b, b)
    out_wait(0)
    out_wait(1)


_sc_kernel = functools.partial(
    pl.kernel,
    mesh=plsc.VectorSubcoreMesh(core_axis_name="c", subcore_axis_name="s"),
    out_type=jax.ShapeDtypeStruct((N_TOTAL,), jnp.float32),
    compiler_params=pltpu.CompilerParams(needs_layout_passes=False),
    scratch_types=[
        pltpu.VMEM((TBL,), jnp.float32),
        pltpu.VMEM((TBL,), jnp.float32),
        pltpu.VMEM((TBL,), jnp.float32),
        pltpu.VMEM((TBL,), jnp.float32),
        pltpu.VMEM((L,), jnp.float32),
        pltpu.VMEM((TBL,), jnp.int32),
        pltpu.VMEM((CHUNK,), jnp.float32),
        pltpu.VMEM((CHUNK,), jnp.float32),
        pltpu.VMEM((CHUNK,), jnp.float32),
        pltpu.VMEM((CHUNK,), jnp.float32),
        pltpu.SemaphoreType.DMA,
        pltpu.SemaphoreType.DMA,
        pltpu.SemaphoreType.DMA,
        pltpu.SemaphoreType.DMA,
    ],
)(_sc_body)


def kernel(x, knots, a, b, c, d):
    return _sc_kernel(x, knots, a, b, c, d)
